# baseline (speedup 1.0000x reference)
import jax
import jax.numpy as jnp
from jax.experimental import pallas as pl
from jax.experimental.pallas import tpu as pltpu

_N = 4096
_D = 64
_K = 4
_BN = 1024
_NBUF = 3


def _mixed_op_body(mask_hbm, h_ref, w_ref, out_hbm, hc_ref):
    acc = jnp.zeros((_N, _D), jnp.float32)
    c = jnp.float32(0.0)
    for k in range(_K):
        wk = w_ref[k]
        acc = acc + jnp.where(wk > 0, wk, 0.0) * h_ref[k]
        c = c + jnp.where(wk > 0, jnp.float32(0.0), wk)
    hc_ref[...] = acc.astype(jnp.bfloat16)

    def _inner(mask_blk, out_blk):
        out_blk[...] = (
            jnp.dot(
                mask_blk[...].astype(jnp.bfloat16),
                hc_ref[...],
                preferred_element_type=jnp.float32,
            )
            + c
        )

    pltpu.emit_pipeline(
        _inner,
        grid=(_N // _BN,),
        in_specs=[
            pl.BlockSpec(
                (_BN, _N),
                lambda i: (i, 0),
                pipeline_mode=pl.Buffered(buffer_count=_NBUF),
            )
        ],
        out_specs=[pl.BlockSpec((_BN, _D), lambda i: (i, 0))],
    )(mask_hbm, out_hbm)


@jax.jit
def kernel(mask_matrix, h_op_list, weights):
    return pl.pallas_call(
        _mixed_op_body,
        in_specs=[
            pl.BlockSpec(memory_space=pltpu.HBM),
            pl.BlockSpec((_K, _N, _D), lambda: (0, 0, 0)),
            pl.BlockSpec(memory_space=pltpu.SMEM),
        ],
        out_specs=pl.BlockSpec(memory_space=pltpu.HBM),
        out_shape=jax.ShapeDtypeStruct((_N, _D), jnp.float32),
        scratch_shapes=[
            pltpu.VMEM((_N, _D), jnp.bfloat16),
        ],
    )(mask_matrix, h_op_list, weights)


# speedup vs baseline: 1.0157x; 1.0157x over previous
import jax
import jax.numpy as jnp
from jax.experimental import pallas as pl
from jax.experimental.pallas import tpu as pltpu

_N = 4096
_D = 64
_K = 4
_BK = 1024
_NSTEP = _N // _BK


def _mixed_op_body(mask_ref, h_ref, w_ref, out_ref):
    j = pl.program_id(0)

    acc = jnp.zeros((_BK, _D), jnp.float32)
    c = jnp.float32(0.0)
    for k in range(_K):
        wk = w_ref[k]
        acc = acc + jnp.where(wk > 0, wk, 0.0) * h_ref[k]
        c = c + jnp.where(wk > 0, jnp.float32(0.0), wk)

    partial = jnp.dot(
        mask_ref[...].astype(jnp.bfloat16),
        acc.astype(jnp.bfloat16),
        preferred_element_type=jnp.float32,
    )

    @pl.when(j == 0)
    def _init():
        out_ref[...] = partial

    @pl.when(j > 0)
    def _accum():
        out_ref[...] += partial

    @pl.when(j == _NSTEP - 1)
    def _final():
        out_ref[...] += c


@jax.jit
def kernel(mask_matrix, h_op_list, weights):
    return pl.pallas_call(
        _mixed_op_body,
        grid=(_NSTEP,),
        in_specs=[
            pl.BlockSpec((_N, _BK), lambda j: (0, j)),
            pl.BlockSpec((_K, _BK, _D), lambda j: (0, j, 0)),
            pl.BlockSpec(memory_space=pltpu.SMEM),
        ],
        out_specs=pl.BlockSpec((_N, _D), lambda j: (0, 0)),
        out_shape=jax.ShapeDtypeStruct((_N, _D), jnp.float32),
    )(mask_matrix, h_op_list, weights)


# speedup vs baseline: 1.0178x; 1.0021x over previous
import jax
import jax.numpy as jnp
from jax.experimental import pallas as pl
from jax.experimental.pallas import tpu as pltpu

_N = 4096
_D = 64
_K = 4
_CH = 256
_NSTEP = _N // _CH
_NBUF = 10
_LOOK = _NBUF - 2


def _chunk_copy(mask_hbm, mbuf, sems, chunk, slot):
    return pltpu.make_async_copy(
        mask_hbm.at[pl.ds(chunk * _CH, _CH), :],
        mbuf.at[slot],
        sems.at[slot],
    )


def _chunk_start(mask_hbm, mbuf, sems, chunk, slot, prio):
    pltpu.async_copy(
        mask_hbm.at[pl.ds(chunk * _CH, _CH), :],
        mbuf.at[slot],
        sems.at[slot],
        priority=prio,
    )


def _mixed_op_body(mask_hbm, h_ref, w_ref, out_ref, mbuf, hc_ref, sems):
    i = pl.program_id(0)

    @pl.when(i == 0)
    def _prologue():
        for j in range(_LOOK):
            _chunk_start(mask_hbm, mbuf, sems, j, j, j % 2)
        acc = jnp.zeros((_N, _D), jnp.float32)
        for k in range(_K):
            wk = w_ref[k]
            acc = acc + jnp.where(wk > 0, wk, 0.0) * h_ref[k]
        hc_ref[...] = acc.astype(jnp.bfloat16)

    _chunk_copy(mask_hbm, mbuf, sems, i, i % _NBUF).wait()

    nxt = i + _LOOK

    @pl.when(jnp.logical_and(nxt < _NSTEP, nxt % 2 == 0))
    def _prefetch_even():
        _chunk_start(mask_hbm, mbuf, sems, nxt, nxt % _NBUF, 0)

    @pl.when(jnp.logical_and(nxt < _NSTEP, nxt % 2 == 1))
    def _prefetch_odd():
        _chunk_start(mask_hbm, mbuf, sems, nxt, nxt % _NBUF, 1)

    c = jnp.float32(0.0)
    for k in range(_K):
        wk = w_ref[k]
        c = c + jnp.where(wk > 0, jnp.float32(0.0), wk)
    out_ref[...] = (
        jnp.dot(
            mbuf[i % _NBUF].astype(jnp.bfloat16),
            hc_ref[...],
            preferred_element_type=jnp.float32,
        )
        + c
    )


@jax.jit
def kernel(mask_matrix, h_op_list, weights):
    return pl.pallas_call(
        _mixed_op_body,
        grid=(_NSTEP,),
        in_specs=[
            pl.BlockSpec(memory_space=pltpu.HBM),
            pl.BlockSpec((_K, _N, _D), lambda i: (0, 0, 0)),
            pl.BlockSpec(memory_space=pltpu.SMEM),
        ],
        out_specs=pl.BlockSpec((_CH, _D), lambda i: (i, 0)),
        out_shape=jax.ShapeDtypeStruct((_N, _D), jnp.float32),
        scratch_shapes=[
            pltpu.VMEM((_NBUF, _CH, _N), jnp.float32),
            pltpu.VMEM((_N, _D), jnp.bfloat16),
            pltpu.SemaphoreType.DMA((_NBUF,)),
        ],
    )(mask_matrix, h_op_list, weights)


# speedup vs baseline: 1.0752x; 1.0564x over previous
import jax
import jax.numpy as jnp
from jax.experimental import pallas as pl
from jax.experimental.pallas import tpu as pltpu

_N = 4096
_D = 64
_K = 4
_BN = 512


def _mixed_op_body(mask_ref, h_ref, w_ref, out_ref, hc_ref):
    @pl.when(pl.program_id(0) == 0)
    def _combine():
        acc = jnp.zeros((_N, _D), jnp.float32)
        for k in range(_K):
            wk = w_ref[k]
            acc = acc + jnp.where(wk > 0, wk, 0.0) * h_ref[pl.ds(k * _N, _N), :]
        hc_ref[...] = acc.astype(jnp.bfloat16)

    c = jnp.float32(0.0)
    for k in range(_K):
        wk = w_ref[k]
        c = c + jnp.where(wk > 0, jnp.float32(0.0), wk)
    out_ref[...] = (
        jnp.dot(
            mask_ref[...].astype(jnp.bfloat16),
            hc_ref[...],
            preferred_element_type=jnp.float32,
        )
        + c
    )


@jax.jit
def kernel(mask_matrix, h_op_list, weights):
    h2 = jnp.reshape(h_op_list, (_K * _N, _D))
    return pl.pallas_call(
        _mixed_op_body,
        grid=(_N // _BN,),
        in_specs=[
            pl.BlockSpec((_BN, _N), lambda i: (i, 0)),
            pl.BlockSpec((_K * _N, _D), lambda i: (0, 0)),
            pl.BlockSpec(memory_space=pltpu.SMEM),
        ],
        out_specs=pl.BlockSpec((_BN, _D), lambda i: (i, 0)),
        out_shape=jax.ShapeDtypeStruct((_N, _D), jnp.float32),
        scratch_shapes=[
            pltpu.VMEM((_N, _D), jnp.bfloat16),
        ],
    )(mask_matrix, h2, weights)
